# baseline (speedup 1.0000x reference)
import functools

import jax
import jax.numpy as jnp
from jax import lax
from jax.experimental import pallas as pl
from jax.experimental.pallas import tpu as pltpu
from jax.experimental.pallas import tpu_sc as plsc

EMBED_D = 128
C_LEN = 100
C_PAD = 112
B3 = 3072
NUM_CORES = 2
NUM_SUBCORES = 16
NW = NUM_CORES * NUM_SUBCORES
BPW = B3 // NW
NB = 4
T_BLK = 10
T_OUTER = C_LEN // T_BLK


def _sc_gather_body(ids_hbm, pcontent_hbm, wembed_hbm, emb_hbm,
                    ids_v, tok_v, idx_v, buf0, buf1, buf2, buf3, sg, ss):
    bufs = (buf0, buf1, buf2, buf3)
    wid = lax.axis_index("s") * NUM_CORES + lax.axis_index("c")
    base = wid * BPW
    pltpu.sync_copy(ids_hbm.at[pl.ds(base, BPW)], ids_v)
    pltpu.async_copy(pcontent_hbm.at[ids_v], tok_v, sg.at[0]).wait()

    iota = lax.iota(jnp.int32, 16)
    for j in range(NB):
        for i0 in range(C_PAD // 16):
            lane = iota + (i0 * 16)
            idx_v[j, pl.ds(i0 * 16, 16)] = lane * B3 + (base + j - NB)

    def tok_idx(b):
        return tok_v.at[b, pl.ds(0, C_PAD)]

    def fire_scatter(slot):
        for i0 in range(C_PAD // 16):
            idx_v[slot, pl.ds(i0 * 16, 16)] = (
                idx_v[slot, pl.ds(i0 * 16, 16)] + NB)
        pltpu.async_copy(bufs[slot], emb_hbm.at[idx_v.at[slot]], ss.at[slot])

    def drain_scatter(slot):
        pltpu.make_async_copy(
            bufs[slot], emb_hbm.at[pl.ds(0, C_PAD)], ss.at[slot]).wait()

    for j in range(2):
        pltpu.async_copy(wembed_hbm.at[tok_idx(j)], bufs[j], sg.at[j])

    def round_body(g, carry):
        for j in range(NB):
            b = g * NB + j
            pltpu.make_async_copy(
                wembed_hbm.at[tok_idx(b)], bufs[j], sg.at[j]).wait()
            fire_scatter(j)
            f = b + 2
            fs = (j + 2) % NB

            @pl.when(f < BPW)
            def _fire():
                @pl.when(b >= 2)
                def _drain():
                    drain_scatter(fs)
                pltpu.async_copy(
                    wembed_hbm.at[tok_idx(f)], bufs[fs], sg.at[fs])
        return carry

    lax.fori_loop(0, BPW // NB, round_body, 0)
    for j in range(NB):
        drain_scatter(j)


def _sc_gather(flat_ids, p_content_pad, word_embed):
    mesh = plsc.VectorSubcoreMesh(
        core_axis_name="c", subcore_axis_name="s",
        num_cores=NUM_CORES, num_subcores=NUM_SUBCORES)
    run = pl.kernel(
        _sc_gather_body,
        out_type=jax.ShapeDtypeStruct((C_PAD * B3, EMBED_D), jnp.float32),
        mesh=mesh,
        scratch_types=[
            pltpu.VMEM((BPW,), jnp.int32),
            pltpu.VMEM((BPW, 128), jnp.int32),
            pltpu.VMEM((NB, C_PAD), jnp.int32),
            pltpu.VMEM((C_PAD, EMBED_D), jnp.float32),
            pltpu.VMEM((C_PAD, EMBED_D), jnp.float32),
            pltpu.VMEM((C_PAD, EMBED_D), jnp.float32),
            pltpu.VMEM((C_PAD, EMBED_D), jnp.float32),
            pltpu.SemaphoreType.DMA((NB,)),
            pltpu.SemaphoreType.DMA((NB,)),
        ],
    )
    return run(flat_ids, p_content_pad, word_embed)


def _lstm_body(emb_ref, wih_ref, whh_ref, bias_ref, out_ref, h_ref, c_ref):
    t = pl.program_id(0)

    @pl.when(t == 0)
    def _init():
        h_ref[...] = jnp.zeros_like(h_ref)
        c_ref[...] = jnp.zeros_like(c_ref)
        out_ref[...] = jnp.zeros_like(out_ref)

    acc = out_ref[...]
    h = h_ref[...]
    c = c_ref[...]
    for k in range(T_BLK):
        x = emb_ref[pl.ds(k * B3, B3), :].astype(jnp.bfloat16)
        gates = (
            jnp.dot(x, wih_ref[...], preferred_element_type=jnp.float32)
            + jnp.dot(h.astype(jnp.bfloat16), whh_ref[...],
                      preferred_element_type=jnp.float32)
            + bias_ref[0:1, :]
        )
        i = 0.5 * jnp.tanh(gates[:, 0:EMBED_D]) + 0.5
        f = 0.5 * jnp.tanh(gates[:, EMBED_D:2 * EMBED_D]) + 0.5
        g = jnp.tanh(gates[:, 2 * EMBED_D:3 * EMBED_D])
        o = 0.5 * jnp.tanh(gates[:, 3 * EMBED_D:4 * EMBED_D]) + 0.5
        c = f * c + i * g
        h = o * jnp.tanh(c)
        acc += h
    h_ref[...] = h
    c_ref[...] = c
    out_ref[...] = acc

    @pl.when(t == T_OUTER - 1)
    def _finish():
        out_ref[...] = acc * (1.0 / C_LEN)


def _lstm(emb_tm, wih_t, whh_t, bias):
    return pl.pallas_call(
        _lstm_body,
        grid=(T_OUTER,),
        in_specs=[
            pl.BlockSpec((T_BLK * B3, EMBED_D), lambda t: (t, 0)),
            pl.BlockSpec((EMBED_D, 4 * EMBED_D), lambda t: (0, 0)),
            pl.BlockSpec((EMBED_D, 4 * EMBED_D), lambda t: (0, 0)),
            pl.BlockSpec((8, 4 * EMBED_D), lambda t: (0, 0)),
        ],
        out_specs=pl.BlockSpec((B3, EMBED_D), lambda t: (0, 0)),
        out_shape=jax.ShapeDtypeStruct((B3, EMBED_D), jnp.float32),
        scratch_shapes=[
            pltpu.VMEM((B3, EMBED_D), jnp.float32),
            pltpu.VMEM((B3, EMBED_D), jnp.float32),
        ],
        compiler_params=pltpu.CompilerParams(
            vmem_limit_bytes=100 * 1024 * 1024),
    )(emb_tm, wih_t, whh_t, bias)


def kernel(triple_batch, triple_index, word_embed, p_content, W_ih, W_hh,
           b_ih, b_hh):
    flat_ids = jnp.transpose(triple_batch.astype(jnp.int32)).reshape(B3)
    p_content_pad = jnp.pad(
        p_content.astype(jnp.int32), ((0, 0), (0, 128 - C_LEN)))
    emb_tm = _sc_gather(flat_ids, p_content_pad, word_embed)

    gate_scale = jnp.concatenate([
        jnp.full((2 * EMBED_D,), 0.5, jnp.float32),
        jnp.ones((EMBED_D,), jnp.float32),
        jnp.full((EMBED_D,), 0.5, jnp.float32),
    ])
    wih_t = (jnp.transpose(W_ih) * gate_scale[None, :]).astype(jnp.bfloat16)
    whh_t = (jnp.transpose(W_hh) * gate_scale[None, :]).astype(jnp.bfloat16)
    bias = jnp.broadcast_to(
        ((b_ih + b_hh) * gate_scale)[None, :], (8, 4 * EMBED_D))
    out = _lstm(emb_tm, wih_t, whh_t, bias)
    return (out[0:1024], out[1024:2048], out[2048:3072])


# speedup vs baseline: 5.1016x; 5.1016x over previous
import functools

import jax
import jax.numpy as jnp
from jax import lax
from jax.experimental import pallas as pl
from jax.experimental.pallas import tpu as pltpu
from jax.experimental.pallas import tpu_sc as plsc

EMBED_D = 128
C_LEN = 100
B3 = 3072
NUM_CORES = 2
NUM_SUBCORES = 16
NW = NUM_CORES * NUM_SUBCORES
BPW = B3 // NW
NB = 4
T_BLK = 10
T_OUTER = C_LEN // T_BLK


def _sc_gather_body(ids_hbm, pcontent_hbm, wembed_hbm, emb_hbm,
                    ids_v, tok_v, tok_t, buf0, buf1, buf2, buf3, sg, ss):
    bufs = (buf0, buf1, buf2, buf3)
    wid = lax.axis_index("s") * NUM_CORES + lax.axis_index("c")
    base = wid * BPW
    pltpu.sync_copy(ids_hbm.at[pl.ds(base, BPW)], ids_v)
    pltpu.async_copy(pcontent_hbm.at[ids_v], tok_v, sg.at[0]).wait()

    iota = lax.iota(jnp.int32, 16)

    def transpose_row(t, carry):
        for i0 in range(BPW // 16):
            rows = iota + (i0 * 16)
            cols = iota * 0 + t
            tok_t[t, pl.ds(i0 * 16, 16)] = plsc.load_gather(
                tok_v, [rows, cols])
        return carry

    lax.fori_loop(0, C_LEN, transpose_row, 0)

    def gather_t(t, slot):
        pltpu.async_copy(
            wembed_hbm.at[tok_t.at[t, pl.ds(0, BPW)]], bufs[slot],
            sg.at[slot])

    def wait_gather(t, slot):
        pltpu.make_async_copy(
            wembed_hbm.at[tok_t.at[t, pl.ds(0, BPW)]], bufs[slot],
            sg.at[slot]).wait()

    def fire_scatter(t, slot):
        pltpu.async_copy(
            bufs[slot], emb_hbm.at[pl.ds(t * B3 + base, BPW)], ss.at[slot])

    def drain_scatter(slot):
        pltpu.make_async_copy(
            bufs[slot], emb_hbm.at[pl.ds(0, BPW)], ss.at[slot]).wait()

    for j in range(2):
        gather_t(j, j)

    def round_body(g, carry):
        for j in range(NB):
            t = g * NB + j
            wait_gather(t, j)
            fire_scatter(t, j)
            f = t + 2
            fs = (j + 2) % NB

            @pl.when(f < C_LEN)
            def _fire():
                @pl.when(t >= 2)
                def _drain():
                    drain_scatter(fs)
                gather_t(f, fs)
        return carry

    lax.fori_loop(0, C_LEN // NB, round_body, 0)
    for j in range(NB):
        drain_scatter(j)


def _sc_gather(flat_ids, p_content_pad, word_embed):
    mesh = plsc.VectorSubcoreMesh(
        core_axis_name="c", subcore_axis_name="s",
        num_cores=NUM_CORES, num_subcores=NUM_SUBCORES)
    run = pl.kernel(
        _sc_gather_body,
        out_type=jax.ShapeDtypeStruct((C_LEN * B3, EMBED_D), jnp.float32),
        mesh=mesh,
        scratch_types=[
            pltpu.VMEM((BPW,), jnp.int32),
            pltpu.VMEM((BPW, 128), jnp.int32),
            pltpu.VMEM((C_LEN, BPW), jnp.int32),
            pltpu.VMEM((BPW, EMBED_D), jnp.float32),
            pltpu.VMEM((BPW, EMBED_D), jnp.float32),
            pltpu.VMEM((BPW, EMBED_D), jnp.float32),
            pltpu.VMEM((BPW, EMBED_D), jnp.float32),
            pltpu.SemaphoreType.DMA((NB,)),
            pltpu.SemaphoreType.DMA((NB,)),
        ],
        compiler_params=pltpu.CompilerParams(needs_layout_passes=False),
    )
    return run(flat_ids, p_content_pad, word_embed)


def _lstm_body(emb_ref, wih_ref, whh_ref, bias_ref, out_ref, h_ref, c_ref):
    t = pl.program_id(0)

    @pl.when(t == 0)
    def _init():
        h_ref[...] = jnp.zeros_like(h_ref)
        c_ref[...] = jnp.zeros_like(c_ref)
        out_ref[...] = jnp.zeros_like(out_ref)

    acc = out_ref[...]
    h = h_ref[...]
    c = c_ref[...]
    for k in range(T_BLK):
        x = emb_ref[pl.ds(k * B3, B3), :].astype(jnp.bfloat16)
        gates = (
            jnp.dot(x, wih_ref[...], preferred_element_type=jnp.float32)
            + jnp.dot(h.astype(jnp.bfloat16), whh_ref[...],
                      preferred_element_type=jnp.float32)
            + bias_ref[0:1, :]
        )
        i = 0.5 * jnp.tanh(gates[:, 0:EMBED_D]) + 0.5
        f = 0.5 * jnp.tanh(gates[:, EMBED_D:2 * EMBED_D]) + 0.5
        g = jnp.tanh(gates[:, 2 * EMBED_D:3 * EMBED_D])
        o = 0.5 * jnp.tanh(gates[:, 3 * EMBED_D:4 * EMBED_D]) + 0.5
        c = f * c + i * g
        h = o * jnp.tanh(c)
        acc += h
    h_ref[...] = h
    c_ref[...] = c
    out_ref[...] = acc

    @pl.when(t == T_OUTER - 1)
    def _finish():
        out_ref[...] = acc * (1.0 / C_LEN)


def _lstm(emb_tm, wih_t, whh_t, bias):
    return pl.pallas_call(
        _lstm_body,
        grid=(T_OUTER,),
        in_specs=[
            pl.BlockSpec((T_BLK * B3, EMBED_D), lambda t: (t, 0)),
            pl.BlockSpec((EMBED_D, 4 * EMBED_D), lambda t: (0, 0)),
            pl.BlockSpec((EMBED_D, 4 * EMBED_D), lambda t: (0, 0)),
            pl.BlockSpec((8, 4 * EMBED_D), lambda t: (0, 0)),
        ],
        out_specs=pl.BlockSpec((B3, EMBED_D), lambda t: (0, 0)),
        out_shape=jax.ShapeDtypeStruct((B3, EMBED_D), jnp.float32),
        scratch_shapes=[
            pltpu.VMEM((B3, EMBED_D), jnp.float32),
            pltpu.VMEM((B3, EMBED_D), jnp.float32),
        ],
        compiler_params=pltpu.CompilerParams(
            vmem_limit_bytes=100 * 1024 * 1024),
    )(emb_tm, wih_t, whh_t, bias)


def kernel(triple_batch, triple_index, word_embed, p_content, W_ih, W_hh,
           b_ih, b_hh):
    flat_ids = jnp.transpose(triple_batch.astype(jnp.int32)).reshape(B3)
    p_content_pad = jnp.pad(
        p_content.astype(jnp.int32), ((0, 0), (0, 128 - C_LEN)))
    emb_tm = _sc_gather(flat_ids, p_content_pad, word_embed)

    gate_scale = jnp.concatenate([
        jnp.full((2 * EMBED_D,), 0.5, jnp.float32),
        jnp.ones((EMBED_D,), jnp.float32),
        jnp.full((EMBED_D,), 0.5, jnp.float32),
    ])
    wih_t = (jnp.transpose(W_ih) * gate_scale[None, :]).astype(jnp.bfloat16)
    whh_t = (jnp.transpose(W_hh) * gate_scale[None, :]).astype(jnp.bfloat16)
    bias = jnp.broadcast_to(
        ((b_ih + b_hh) * gate_scale)[None, :], (8, 4 * EMBED_D))
    out = _lstm(emb_tm, wih_t, whh_t, bias)
    return (out[0:1024], out[1024:2048], out[2048:3072])


# speedup vs baseline: 5.8639x; 1.1494x over previous
import functools

import jax
import jax.numpy as jnp
from jax import lax
from jax.experimental import pallas as pl
from jax.experimental.pallas import tpu as pltpu
from jax.experimental.pallas import tpu_sc as plsc

EMBED_D = 128
C_LEN = 100
B3 = 3072
NUM_CORES = 2
NUM_SUBCORES = 16
NW = NUM_CORES * NUM_SUBCORES
NB = 4
T_BLK = 10
T_OUTER = C_LEN // T_BLK
N_CHUNK = 2
BC = B3 // N_CHUNK
BPW = BC // NW


def _sc_gather_body(ids_hbm, pcontent_hbm, wembed_hbm, emb_hbm,
                    ids_v, tok_v, tok_t, buf0, buf1, buf2, buf3, sg, ss):
    bufs = (buf0, buf1, buf2, buf3)
    wid = lax.axis_index("s") * NUM_CORES + lax.axis_index("c")
    base = wid * BPW
    pltpu.sync_copy(ids_hbm.at[pl.ds(base, BPW)], ids_v)
    pltpu.async_copy(pcontent_hbm.at[ids_v], tok_v, sg.at[0]).wait()

    iota = lax.iota(jnp.int32, 16)

    def transpose_row(t, carry):
        for i0 in range(BPW // 16):
            rows = iota + (i0 * 16)
            cols = iota * 0 + t
            tok_t[t, pl.ds(i0 * 16, 16)] = plsc.load_gather(
                tok_v, [rows, cols])
        return carry

    lax.fori_loop(0, C_LEN, transpose_row, 0)

    def gather_t(t, slot):
        pltpu.async_copy(
            wembed_hbm.at[tok_t.at[t, pl.ds(0, BPW)]], bufs[slot],
            sg.at[slot])

    def wait_gather(t, slot):
        pltpu.make_async_copy(
            wembed_hbm.at[tok_t.at[t, pl.ds(0, BPW)]], bufs[slot],
            sg.at[slot]).wait()

    def fire_scatter(t, slot):
        pltpu.async_copy(
            bufs[slot], emb_hbm.at[pl.ds(t * BC + base, BPW)], ss.at[slot])

    def drain_scatter(slot):
        pltpu.make_async_copy(
            bufs[slot], emb_hbm.at[pl.ds(0, BPW)], ss.at[slot]).wait()

    for j in range(2):
        gather_t(j, j)

    def round_body(g, carry):
        for j in range(NB):
            t = g * NB + j
            wait_gather(t, j)
            fire_scatter(t, j)
            f = t + 2
            fs = (j + 2) % NB

            @pl.when(f < C_LEN)
            def _fire():
                @pl.when(t >= 2)
                def _drain():
                    drain_scatter(fs)
                gather_t(f, fs)
        return carry

    lax.fori_loop(0, C_LEN // NB, round_body, 0)
    for j in range(NB):
        drain_scatter(j)


def _sc_gather(flat_ids, p_content_pad, word_embed):
    mesh = plsc.VectorSubcoreMesh(
        core_axis_name="c", subcore_axis_name="s",
        num_cores=NUM_CORES, num_subcores=NUM_SUBCORES)
    run = pl.kernel(
        _sc_gather_body,
        out_type=jax.ShapeDtypeStruct((C_LEN * BC, EMBED_D), jnp.float32),
        mesh=mesh,
        scratch_types=[
            pltpu.VMEM((BPW,), jnp.int32),
            pltpu.VMEM((BPW, 128), jnp.int32),
            pltpu.VMEM((C_LEN, BPW), jnp.int32),
            pltpu.VMEM((BPW, EMBED_D), jnp.float32),
            pltpu.VMEM((BPW, EMBED_D), jnp.float32),
            pltpu.VMEM((BPW, EMBED_D), jnp.float32),
            pltpu.VMEM((BPW, EMBED_D), jnp.float32),
            pltpu.SemaphoreType.DMA((NB,)),
            pltpu.SemaphoreType.DMA((NB,)),
        ],
        compiler_params=pltpu.CompilerParams(needs_layout_passes=False),
    )
    return run(flat_ids, p_content_pad, word_embed)


def _lstm_body(emb_ref, wih_ref, whh_ref, bias_ref, out_ref, h_ref, c_ref):
    t = pl.program_id(0)

    @pl.when(t == 0)
    def _init():
        h_ref[...] = jnp.zeros_like(h_ref)
        c_ref[...] = jnp.zeros_like(c_ref)
        out_ref[...] = jnp.zeros_like(out_ref)

    acc = out_ref[...]
    h = h_ref[...]
    c = c_ref[...]
    for k in range(T_BLK):
        x = emb_ref[pl.ds(k * BC, BC), :].astype(jnp.bfloat16)
        gates = (
            jnp.dot(x, wih_ref[...], preferred_element_type=jnp.float32)
            + jnp.dot(h.astype(jnp.bfloat16), whh_ref[...],
                      preferred_element_type=jnp.float32)
            + bias_ref[0:1, :]
        )
        i = 0.5 * jnp.tanh(gates[:, 0:EMBED_D]) + 0.5
        f = 0.5 * jnp.tanh(gates[:, EMBED_D:2 * EMBED_D]) + 0.5
        g = jnp.tanh(gates[:, 2 * EMBED_D:3 * EMBED_D])
        o = 0.5 * jnp.tanh(gates[:, 3 * EMBED_D:4 * EMBED_D]) + 0.5
        c = f * c + i * g
        h = o * jnp.tanh(c)
        acc += h
    h_ref[...] = h
    c_ref[...] = c
    out_ref[...] = acc

    @pl.when(t == T_OUTER - 1)
    def _finish():
        out_ref[...] = acc * (1.0 / C_LEN)


def _lstm(emb_tm, wih_t, whh_t, bias):
    return pl.pallas_call(
        _lstm_body,
        grid=(T_OUTER,),
        in_specs=[
            pl.BlockSpec((T_BLK * BC, EMBED_D), lambda t: (t, 0)),
            pl.BlockSpec((EMBED_D, 4 * EMBED_D), lambda t: (0, 0)),
            pl.BlockSpec((EMBED_D, 4 * EMBED_D), lambda t: (0, 0)),
            pl.BlockSpec((8, 4 * EMBED_D), lambda t: (0, 0)),
        ],
        out_specs=pl.BlockSpec((BC, EMBED_D), lambda t: (0, 0)),
        out_shape=jax.ShapeDtypeStruct((BC, EMBED_D), jnp.float32),
        scratch_shapes=[
            pltpu.VMEM((BC, EMBED_D), jnp.float32),
            pltpu.VMEM((BC, EMBED_D), jnp.float32),
        ],
        compiler_params=pltpu.CompilerParams(
            vmem_limit_bytes=100 * 1024 * 1024),
    )(emb_tm, wih_t, whh_t, bias)


def kernel(triple_batch, triple_index, word_embed, p_content, W_ih, W_hh,
           b_ih, b_hh):
    flat_ids = jnp.transpose(triple_batch.astype(jnp.int32)).reshape(B3)
    p_content_pad = jnp.pad(
        p_content.astype(jnp.int32), ((0, 0), (0, 128 - C_LEN)))

    gate_scale = jnp.concatenate([
        jnp.full((2 * EMBED_D,), 0.5, jnp.float32),
        jnp.ones((EMBED_D,), jnp.float32),
        jnp.full((EMBED_D,), 0.5, jnp.float32),
    ])
    wih_t = (jnp.transpose(W_ih) * gate_scale[None, :]).astype(jnp.bfloat16)
    whh_t = (jnp.transpose(W_hh) * gate_scale[None, :]).astype(jnp.bfloat16)
    bias = jnp.broadcast_to(
        ((b_ih + b_hh) * gate_scale)[None, :], (8, 4 * EMBED_D))

    embs = [
        _sc_gather(flat_ids[c * BC:(c + 1) * BC], p_content_pad, word_embed)
        for c in range(N_CHUNK)
    ]
    outs = [_lstm(emb, wih_t, whh_t, bias) for emb in embs]
    out = jnp.concatenate(outs, axis=0)
    return (out[0:1024], out[1024:2048], out[2048:3072])


# speedup vs baseline: 6.0542x; 1.0324x over previous
import functools

import jax
import jax.numpy as jnp
from jax import lax
from jax.experimental import pallas as pl
from jax.experimental.pallas import tpu as pltpu
from jax.experimental.pallas import tpu_sc as plsc

EMBED_D = 128
C_LEN = 100
B3 = 3072
NUM_CORES = 2
NUM_SUBCORES = 16
NW = NUM_CORES * NUM_SUBCORES
NB = 4
T_BLK = 10
T_OUTER = C_LEN // T_BLK
N_CHUNK = 3
BC = B3 // N_CHUNK
BPW = BC // NW


def _sc_gather_body(ids_hbm, pcontent_hbm, wembed_hbm, emb_hbm,
                    ids_v, tok_v, tok_t, buf0, buf1, buf2, buf3, sg, ss):
    bufs = (buf0, buf1, buf2, buf3)
    wid = lax.axis_index("s") * NUM_CORES + lax.axis_index("c")
    base = wid * BPW
    pltpu.sync_copy(ids_hbm.at[pl.ds(base, BPW)], ids_v)
    pltpu.async_copy(pcontent_hbm.at[ids_v], tok_v, sg.at[0]).wait()

    iota = lax.iota(jnp.int32, 16)

    def transpose_row(t, carry):
        for i0 in range(BPW // 16):
            rows = iota + (i0 * 16)
            cols = iota * 0 + t
            tok_t[t, pl.ds(i0 * 16, 16)] = plsc.load_gather(
                tok_v, [rows, cols])
        return carry

    lax.fori_loop(0, C_LEN, transpose_row, 0)

    def gather_t(t, slot):
        pltpu.async_copy(
            wembed_hbm.at[tok_t.at[t, pl.ds(0, BPW)]], bufs[slot],
            sg.at[slot])

    def wait_gather(t, slot):
        pltpu.make_async_copy(
            wembed_hbm.at[tok_t.at[t, pl.ds(0, BPW)]], bufs[slot],
            sg.at[slot]).wait()

    def fire_scatter(t, slot):
        pltpu.async_copy(
            bufs[slot], emb_hbm.at[pl.ds(t * BC + base, BPW)], ss.at[slot])

    def drain_scatter(slot):
        pltpu.make_async_copy(
            bufs[slot], emb_hbm.at[pl.ds(0, BPW)], ss.at[slot]).wait()

    for j in range(2):
        gather_t(j, j)

    def round_body(g, carry):
        for j in range(NB):
            t = g * NB + j
            wait_gather(t, j)
            fire_scatter(t, j)
            f = t + 2
            fs = (j + 2) % NB

            @pl.when(f < C_LEN)
            def _fire():
                @pl.when(t >= 2)
                def _drain():
                    drain_scatter(fs)
                gather_t(f, fs)
        return carry

    lax.fori_loop(0, C_LEN // NB, round_body, 0)
    for j in range(NB):
        drain_scatter(j)


def _sc_gather(flat_ids, p_content_pad, word_embed):
    mesh = plsc.VectorSubcoreMesh(
        core_axis_name="c", subcore_axis_name="s",
        num_cores=NUM_CORES, num_subcores=NUM_SUBCORES)
    run = pl.kernel(
        _sc_gather_body,
        out_type=jax.ShapeDtypeStruct((C_LEN * BC, EMBED_D), jnp.float32),
        mesh=mesh,
        scratch_types=[
            pltpu.VMEM((BPW,), jnp.int32),
            pltpu.VMEM((BPW, 128), jnp.int32),
            pltpu.VMEM((C_LEN, BPW), jnp.int32),
            pltpu.VMEM((BPW, EMBED_D), jnp.float32),
            pltpu.VMEM((BPW, EMBED_D), jnp.float32),
            pltpu.VMEM((BPW, EMBED_D), jnp.float32),
            pltpu.VMEM((BPW, EMBED_D), jnp.float32),
            pltpu.SemaphoreType.DMA((NB,)),
            pltpu.SemaphoreType.DMA((NB,)),
        ],
        compiler_params=pltpu.CompilerParams(needs_layout_passes=False),
    )
    return run(flat_ids, p_content_pad, word_embed)


def _lstm_body(emb_ref, wih_ref, whh_ref, bias_ref, out_ref, h_ref, c_ref):
    t = pl.program_id(0)

    @pl.when(t == 0)
    def _init():
        h_ref[...] = jnp.zeros_like(h_ref)
        c_ref[...] = jnp.zeros_like(c_ref)
        out_ref[...] = jnp.zeros_like(out_ref)

    acc = out_ref[...]
    h = h_ref[...]
    c = c_ref[...]
    for k in range(T_BLK):
        x = emb_ref[pl.ds(k * BC, BC), :].astype(jnp.bfloat16)
        gates = (
            jnp.dot(x, wih_ref[...], preferred_element_type=jnp.float32)
            + jnp.dot(h.astype(jnp.bfloat16), whh_ref[...],
                      preferred_element_type=jnp.float32)
            + bias_ref[0:1, :]
        )
        i = 0.5 * jnp.tanh(gates[:, 0:EMBED_D]) + 0.5
        f = 0.5 * jnp.tanh(gates[:, EMBED_D:2 * EMBED_D]) + 0.5
        g = jnp.tanh(gates[:, 2 * EMBED_D:3 * EMBED_D])
        o = 0.5 * jnp.tanh(gates[:, 3 * EMBED_D:4 * EMBED_D]) + 0.5
        c = f * c + i * g
        h = o * jnp.tanh(c)
        acc += h
    h_ref[...] = h
    c_ref[...] = c
    out_ref[...] = acc

    @pl.when(t == T_OUTER - 1)
    def _finish():
        out_ref[...] = acc * (1.0 / C_LEN)


def _lstm(emb_tm, wih_t, whh_t, bias):
    return pl.pallas_call(
        _lstm_body,
        grid=(T_OUTER,),
        in_specs=[
            pl.BlockSpec((T_BLK * BC, EMBED_D), lambda t: (t, 0)),
            pl.BlockSpec((EMBED_D, 4 * EMBED_D), lambda t: (0, 0)),
            pl.BlockSpec((EMBED_D, 4 * EMBED_D), lambda t: (0, 0)),
            pl.BlockSpec((8, 4 * EMBED_D), lambda t: (0, 0)),
        ],
        out_specs=pl.BlockSpec((BC, EMBED_D), lambda t: (0, 0)),
        out_shape=jax.ShapeDtypeStruct((BC, EMBED_D), jnp.float32),
        scratch_shapes=[
            pltpu.VMEM((BC, EMBED_D), jnp.float32),
            pltpu.VMEM((BC, EMBED_D), jnp.float32),
        ],
        compiler_params=pltpu.CompilerParams(
            vmem_limit_bytes=100 * 1024 * 1024),
    )(emb_tm, wih_t, whh_t, bias)


def kernel(triple_batch, triple_index, word_embed, p_content, W_ih, W_hh,
           b_ih, b_hh):
    flat_ids = jnp.transpose(triple_batch.astype(jnp.int32)).reshape(B3)
    p_content_pad = jnp.pad(
        p_content.astype(jnp.int32), ((0, 0), (0, 128 - C_LEN)))

    gate_scale = jnp.concatenate([
        jnp.full((2 * EMBED_D,), 0.5, jnp.float32),
        jnp.ones((EMBED_D,), jnp.float32),
        jnp.full((EMBED_D,), 0.5, jnp.float32),
    ])
    wih_t = (jnp.transpose(W_ih) * gate_scale[None, :]).astype(jnp.bfloat16)
    whh_t = (jnp.transpose(W_hh) * gate_scale[None, :]).astype(jnp.bfloat16)
    bias = jnp.broadcast_to(
        ((b_ih + b_hh) * gate_scale)[None, :], (8, 4 * EMBED_D))

    embs = [
        _sc_gather(flat_ids[c * BC:(c + 1) * BC], p_content_pad, word_embed)
        for c in range(N_CHUNK)
    ]
    outs = [_lstm(emb, wih_t, whh_t, bias) for emb in embs]
    out = jnp.concatenate(outs, axis=0)
    return (out[0:1024], out[1024:2048], out[2048:3072])


# speedup vs baseline: 6.5266x; 1.0780x over previous
import functools

import jax
import jax.numpy as jnp
from jax import lax
from jax.experimental import pallas as pl
from jax.experimental.pallas import tpu as pltpu
from jax.experimental.pallas import tpu_sc as plsc

EMBED_D = 128
C_LEN = 100
B3 = 3072
NUM_CORES = 2
NUM_SUBCORES = 16
NW = NUM_CORES * NUM_SUBCORES
NB = 4
T_BLK = 20
T_OUTER = C_LEN // T_BLK
N_CHUNK = 3
BC = B3 // N_CHUNK
BPW = BC // NW


def _sc_gather_body(ids_hbm, pcontent_hbm, wembed_hbm, emb_hbm,
                    ids_v, tok_v, tok_t, buf0, buf1, buf2, buf3, sg, ss):
    bufs = (buf0, buf1, buf2, buf3)
    wid = lax.axis_index("s") * NUM_CORES + lax.axis_index("c")
    base = wid * BPW
    pltpu.sync_copy(ids_hbm.at[pl.ds(base, BPW)], ids_v)
    pltpu.async_copy(pcontent_hbm.at[ids_v], tok_v, sg.at[0]).wait()

    iota = lax.iota(jnp.int32, 16)

    def transpose_row(t, carry):
        for i0 in range(BPW // 16):
            rows = iota + (i0 * 16)
            cols = iota * 0 + t
            tok_t[t, pl.ds(i0 * 16, 16)] = plsc.load_gather(
                tok_v, [rows, cols])
        return carry

    lax.fori_loop(0, C_LEN, transpose_row, 0)

    def gather_t(t, slot):
        pltpu.async_copy(
            wembed_hbm.at[tok_t.at[t, pl.ds(0, BPW)]], bufs[slot],
            sg.at[slot])

    def wait_gather(t, slot):
        pltpu.make_async_copy(
            wembed_hbm.at[tok_t.at[t, pl.ds(0, BPW)]], bufs[slot],
            sg.at[slot]).wait()

    def fire_scatter(t, slot):
        pltpu.async_copy(
            bufs[slot], emb_hbm.at[pl.ds(t * BC + base, BPW)], ss.at[slot])

    def drain_scatter(slot):
        pltpu.make_async_copy(
            bufs[slot], emb_hbm.at[pl.ds(0, BPW)], ss.at[slot]).wait()

    for j in range(2):
        gather_t(j, j)

    def round_body(g, carry):
        for j in range(NB):
            t = g * NB + j
            wait_gather(t, j)
            fire_scatter(t, j)
            f = t + 2
            fs = (j + 2) % NB

            @pl.when(f < C_LEN)
            def _fire():
                @pl.when(t >= 2)
                def _drain():
                    drain_scatter(fs)
                gather_t(f, fs)
        return carry

    lax.fori_loop(0, C_LEN // NB, round_body, 0)
    for j in range(NB):
        drain_scatter(j)


def _sc_gather(flat_ids, p_content_pad, word_embed):
    mesh = plsc.VectorSubcoreMesh(
        core_axis_name="c", subcore_axis_name="s",
        num_cores=NUM_CORES, num_subcores=NUM_SUBCORES)
    run = pl.kernel(
        _sc_gather_body,
        out_type=jax.ShapeDtypeStruct((C_LEN * BC, EMBED_D), jnp.float32),
        mesh=mesh,
        scratch_types=[
            pltpu.VMEM((BPW,), jnp.int32),
            pltpu.VMEM((BPW, 128), jnp.int32),
            pltpu.VMEM((C_LEN, BPW), jnp.int32),
            pltpu.VMEM((BPW, EMBED_D), jnp.float32),
            pltpu.VMEM((BPW, EMBED_D), jnp.float32),
            pltpu.VMEM((BPW, EMBED_D), jnp.float32),
            pltpu.VMEM((BPW, EMBED_D), jnp.float32),
            pltpu.SemaphoreType.DMA((NB,)),
            pltpu.SemaphoreType.DMA((NB,)),
        ],
        compiler_params=pltpu.CompilerParams(needs_layout_passes=False),
    )
    return run(flat_ids, p_content_pad, word_embed)


def _lstm_body(emb_ref, wcat_ref, bias_ref, out_ref, h_ref, c_ref):
    t = pl.program_id(0)

    @pl.when(t == 0)
    def _init():
        h_ref[...] = jnp.zeros_like(h_ref)
        c_ref[...] = jnp.zeros_like(c_ref)
        out_ref[...] = jnp.zeros_like(out_ref)

    acc = out_ref[...]
    h = h_ref[...]
    c = c_ref[...]
    for k in range(T_BLK):
        x = emb_ref[pl.ds(k * BC, BC), :].astype(jnp.bfloat16)
        z = jnp.concatenate([x, h.astype(jnp.bfloat16)], axis=1)
        gates = (
            jnp.dot(z, wcat_ref[...], preferred_element_type=jnp.float32)
            + bias_ref[0:1, :]
        )
        th_i = jnp.tanh(gates[:, 0:EMBED_D])
        th_f = jnp.tanh(gates[:, EMBED_D:2 * EMBED_D])
        g = jnp.tanh(gates[:, 2 * EMBED_D:3 * EMBED_D])
        th_o = jnp.tanh(gates[:, 3 * EMBED_D:4 * EMBED_D])
        c = 0.5 * ((1.0 + th_f) * c + (1.0 + th_i) * g)
        h = (1.0 + th_o) * jnp.tanh(c)
        acc += h
    h_ref[...] = h
    c_ref[...] = c
    out_ref[...] = acc

    @pl.when(t == T_OUTER - 1)
    def _finish():
        out_ref[...] = acc * (1.0 / (2 * C_LEN))


def _lstm(emb_tm, wcat, bias):
    return pl.pallas_call(
        _lstm_body,
        grid=(T_OUTER,),
        in_specs=[
            pl.BlockSpec((T_BLK * BC, EMBED_D), lambda t: (t, 0)),
            pl.BlockSpec((2 * EMBED_D, 4 * EMBED_D), lambda t: (0, 0)),
            pl.BlockSpec((8, 4 * EMBED_D), lambda t: (0, 0)),
        ],
        out_specs=pl.BlockSpec((BC, EMBED_D), lambda t: (0, 0)),
        out_shape=jax.ShapeDtypeStruct((BC, EMBED_D), jnp.float32),
        scratch_shapes=[
            pltpu.VMEM((BC, EMBED_D), jnp.float32),
            pltpu.VMEM((BC, EMBED_D), jnp.float32),
        ],
        compiler_params=pltpu.CompilerParams(
            vmem_limit_bytes=100 * 1024 * 1024),
    )(emb_tm, wcat, bias)


def kernel(triple_batch, triple_index, word_embed, p_content, W_ih, W_hh,
           b_ih, b_hh):
    flat_ids = jnp.transpose(triple_batch.astype(jnp.int32)).reshape(B3)
    p_content_pad = jnp.pad(
        p_content.astype(jnp.int32), ((0, 0), (0, 128 - C_LEN)))

    gate_scale = jnp.concatenate([
        jnp.full((2 * EMBED_D,), 0.5, jnp.float32),
        jnp.ones((EMBED_D,), jnp.float32),
        jnp.full((EMBED_D,), 0.5, jnp.float32),
    ])
    wih_t = jnp.transpose(W_ih) * gate_scale[None, :]
    whh_t2 = jnp.transpose(W_hh) * gate_scale[None, :] * 0.5
    wcat = jnp.concatenate([wih_t, whh_t2], axis=0).astype(jnp.bfloat16)
    bias = jnp.broadcast_to(
        ((b_ih + b_hh) * gate_scale)[None, :], (8, 4 * EMBED_D))

    embs = [
        _sc_gather(flat_ids[c * BC:(c + 1) * BC], p_content_pad, word_embed)
        for c in range(N_CHUNK)
    ]
    outs = [_lstm(emb, wcat, bias) for emb in embs]
    out = jnp.concatenate(outs, axis=0)
    return (out[0:1024], out[1024:2048], out[2048:3072])


# speedup vs baseline: 6.5546x; 1.0043x over previous
import functools

import jax
import jax.numpy as jnp
from jax import lax
from jax.experimental import pallas as pl
from jax.experimental.pallas import tpu as pltpu
from jax.experimental.pallas import tpu_sc as plsc

EMBED_D = 128
C_LEN = 100
B3 = 3072
NUM_CORES = 2
NUM_SUBCORES = 16
NW = NUM_CORES * NUM_SUBCORES
NB = 4
T_BLK = 20
T_OUTER = C_LEN // T_BLK
N_CHUNK = 3
BC = B3 // N_CHUNK
BPW = BC // NW


def _sc_gather_body(ids_hbm, pcontent_hbm, wembed_hbm, emb_hbm,
                    ids_v, tok_v, tok_t, buf0, buf1, buf2, buf3, sg, ss):
    bufs = (buf0, buf1, buf2, buf3)
    wid = lax.axis_index("s") * NUM_CORES + lax.axis_index("c")
    base = wid * BPW
    pltpu.sync_copy(ids_hbm.at[pl.ds(base, BPW)], ids_v)
    pltpu.async_copy(pcontent_hbm.at[ids_v], tok_v, sg.at[0]).wait()

    iota = lax.iota(jnp.int32, 16)

    def transpose_row(t, carry):
        for i0 in range(BPW // 16):
            rows = iota + (i0 * 16)
            cols = iota * 0 + t
            tok_t[t, pl.ds(i0 * 16, 16)] = plsc.load_gather(
                tok_v, [rows, cols])
        return carry

    lax.fori_loop(0, C_LEN, transpose_row, 0)

    def gather_t(t, slot):
        pltpu.async_copy(
            wembed_hbm.at[tok_t.at[t, pl.ds(0, BPW)]], bufs[slot],
            sg.at[slot])

    def wait_gather(t, slot):
        pltpu.make_async_copy(
            wembed_hbm.at[tok_t.at[t, pl.ds(0, BPW)]], bufs[slot],
            sg.at[slot]).wait()

    def fire_scatter(t, slot):
        pltpu.async_copy(
            bufs[slot], emb_hbm.at[pl.ds(t * BC + base, BPW)], ss.at[slot])

    def drain_scatter(slot):
        pltpu.make_async_copy(
            bufs[slot], emb_hbm.at[pl.ds(0, BPW)], ss.at[slot]).wait()

    for j in range(2):
        gather_t(j, j)

    def round_body(g, carry):
        for j in range(NB):
            t = g * NB + j
            wait_gather(t, j)
            fire_scatter(t, j)
            f = t + 2
            fs = (j + 2) % NB

            @pl.when(f < C_LEN)
            def _fire():
                @pl.when(t >= 2)
                def _drain():
                    drain_scatter(fs)
                gather_t(f, fs)
        return carry

    lax.fori_loop(0, C_LEN // NB, round_body, 0)
    for j in range(NB):
        drain_scatter(j)


def _sc_gather(flat_ids, p_content_pad, word_embed):
    mesh = plsc.VectorSubcoreMesh(
        core_axis_name="c", subcore_axis_name="s",
        num_cores=NUM_CORES, num_subcores=NUM_SUBCORES)
    run = pl.kernel(
        _sc_gather_body,
        out_type=jax.ShapeDtypeStruct((C_LEN * BC, EMBED_D), jnp.float32),
        mesh=mesh,
        scratch_types=[
            pltpu.VMEM((BPW,), jnp.int32),
            pltpu.VMEM((BPW, 128), jnp.int32),
            pltpu.VMEM((C_LEN, BPW), jnp.int32),
            pltpu.VMEM((BPW, EMBED_D), jnp.float32),
            pltpu.VMEM((BPW, EMBED_D), jnp.float32),
            pltpu.VMEM((BPW, EMBED_D), jnp.float32),
            pltpu.VMEM((BPW, EMBED_D), jnp.float32),
            pltpu.SemaphoreType.DMA((NB,)),
            pltpu.SemaphoreType.DMA((NB,)),
        ],
        compiler_params=pltpu.CompilerParams(needs_layout_passes=False),
    )
    return run(flat_ids, p_content_pad, word_embed)


def _lstm_body(emb_ref, wcat_ref, bias_ref, out_ref, h_ref, c_ref):
    t = pl.program_id(0)

    @pl.when(t == 0)
    def _init():
        h_ref[...] = jnp.zeros_like(h_ref)
        c_ref[...] = jnp.zeros_like(c_ref)
        out_ref[...] = jnp.zeros_like(out_ref)

    acc = out_ref[...]
    h = h_ref[...]
    c = c_ref[...]
    for k in range(T_BLK):
        x = emb_ref[pl.ds(k * BC, BC), :].astype(jnp.bfloat16)
        z = jnp.concatenate([x, h.astype(jnp.bfloat16)], axis=1)
        gates = (
            jnp.dot(z, wcat_ref[...], preferred_element_type=jnp.float32)
            + bias_ref[0:1, :]
        )
        th_i = jnp.tanh(gates[:, 0:EMBED_D])
        th_f = jnp.tanh(gates[:, EMBED_D:2 * EMBED_D])
        g = jnp.tanh(gates[:, 2 * EMBED_D:3 * EMBED_D])
        th_o = jnp.tanh(gates[:, 3 * EMBED_D:4 * EMBED_D])
        c = 0.5 * ((1.0 + th_f) * c + (1.0 + th_i) * g)
        h = (1.0 + th_o) * jnp.tanh(c)
        acc += h
    h_ref[...] = h
    c_ref[...] = c
    out_ref[...] = acc

    @pl.when(t == T_OUTER - 1)
    def _finish():
        out_ref[...] = acc * (1.0 / (2 * C_LEN))


def _lstm(emb_tm, wcat, bias):
    return pl.pallas_call(
        _lstm_body,
        grid=(T_OUTER,),
        in_specs=[
            pl.BlockSpec((T_BLK * BC, EMBED_D), lambda t: (t, 0)),
            pl.BlockSpec((2 * EMBED_D, 4 * EMBED_D), lambda t: (0, 0)),
            pl.BlockSpec((8, 4 * EMBED_D), lambda t: (0, 0)),
        ],
        out_specs=pl.BlockSpec((BC, EMBED_D), lambda t: (0, 0)),
        out_shape=jax.ShapeDtypeStruct((BC, EMBED_D), jnp.float32),
        scratch_shapes=[
            pltpu.VMEM((BC, EMBED_D), jnp.float32),
            pltpu.VMEM((BC, EMBED_D), jnp.float32),
        ],
        compiler_params=pltpu.CompilerParams(
            vmem_limit_bytes=100 * 1024 * 1024),
    )(emb_tm, wcat, bias)


def kernel(triple_batch, triple_index, word_embed, p_content, W_ih, W_hh,
           b_ih, b_hh):
    flat_ids = jnp.transpose(triple_batch.astype(jnp.int32)).reshape(B3)
    p_content_pad = jnp.pad(
        p_content.astype(jnp.int32), ((0, 0), (0, 128 - C_LEN)))

    gate_scale = jnp.concatenate([
        jnp.full((2 * EMBED_D,), 0.5, jnp.float32),
        jnp.ones((EMBED_D,), jnp.float32),
        jnp.full((EMBED_D,), 0.5, jnp.float32),
    ])
    wih_t = jnp.transpose(W_ih) * gate_scale[None, :]
    whh_t2 = jnp.transpose(W_hh) * gate_scale[None, :] * 0.5
    wcat = jnp.concatenate([wih_t, whh_t2], axis=0).astype(jnp.bfloat16)
    bias = jnp.broadcast_to(
        ((b_ih + b_hh) * gate_scale)[None, :], (8, 4 * EMBED_D))

    embs = [
        _sc_gather(flat_ids[c * BC:(c + 1) * BC], p_content_pad, word_embed)
        for c in range(N_CHUNK)
    ]
    outs = [_lstm(emb, wcat, bias) for emb in embs]
    return (outs[0], outs[1], outs[2])


# speedup vs baseline: 7.0019x; 1.0682x over previous
import functools

import jax
import jax.numpy as jnp
from jax import lax
from jax.experimental import pallas as pl
from jax.experimental.pallas import tpu as pltpu
from jax.experimental.pallas import tpu_sc as plsc

EMBED_D = 128
C_LEN = 100
B3 = 3072
NUM_CORES = 2
NUM_SUBCORES = 16
NW = NUM_CORES * NUM_SUBCORES
NB = 4
T_BLK = 20
T_OUTER = C_LEN // T_BLK
N_CHUNK = 2
BC = B3 // N_CHUNK
BPW = BC // NW


def _sc_gather_body(ids_hbm, pcontent_hbm, wembed_hbm, emb_hbm,
                    ids_v, tok_v, tok_t, buf0, buf1, buf2, buf3, sg, ss):
    bufs = (buf0, buf1, buf2, buf3)
    wid = lax.axis_index("s") * NUM_CORES + lax.axis_index("c")
    base = wid * BPW
    pltpu.sync_copy(ids_hbm.at[pl.ds(base, BPW)], ids_v)
    pltpu.async_copy(pcontent_hbm.at[ids_v], tok_v, sg.at[0]).wait()

    iota = lax.iota(jnp.int32, 16)

    def transpose_row(t, carry):
        for i0 in range(BPW // 16):
            rows = iota + (i0 * 16)
            cols = iota * 0 + t
            tok_t[t, pl.ds(i0 * 16, 16)] = plsc.load_gather(
                tok_v, [rows, cols])
        return carry

    lax.fori_loop(0, C_LEN, transpose_row, 0)

    def gather_t(t, slot):
        pltpu.async_copy(
            wembed_hbm.at[tok_t.at[t, pl.ds(0, BPW)]], bufs[slot],
            sg.at[slot])

    def wait_gather(t, slot):
        pltpu.make_async_copy(
            wembed_hbm.at[tok_t.at[t, pl.ds(0, BPW)]], bufs[slot],
            sg.at[slot]).wait()

    def fire_scatter(t, slot):
        pltpu.async_copy(
            bufs[slot], emb_hbm.at[pl.ds(t * BC + base, BPW)], ss.at[slot])

    def drain_scatter(slot):
        pltpu.make_async_copy(
            bufs[slot], emb_hbm.at[pl.ds(0, BPW)], ss.at[slot]).wait()

    for j in range(2):
        gather_t(j, j)

    def round_body(g, carry):
        for j in range(NB):
            t = g * NB + j
            wait_gather(t, j)
            fire_scatter(t, j)
            f = t + 2
            fs = (j + 2) % NB

            @pl.when(f < C_LEN)
            def _fire():
                @pl.when(t >= 2)
                def _drain():
                    drain_scatter(fs)
                gather_t(f, fs)
        return carry

    lax.fori_loop(0, C_LEN // NB, round_body, 0)
    for j in range(NB):
        drain_scatter(j)


def _sc_gather(flat_ids, p_content_pad, word_embed):
    mesh = plsc.VectorSubcoreMesh(
        core_axis_name="c", subcore_axis_name="s",
        num_cores=NUM_CORES, num_subcores=NUM_SUBCORES)
    run = pl.kernel(
        _sc_gather_body,
        out_type=jax.ShapeDtypeStruct((C_LEN * BC, EMBED_D), jnp.float32),
        mesh=mesh,
        scratch_types=[
            pltpu.VMEM((BPW,), jnp.int32),
            pltpu.VMEM((BPW, 128), jnp.int32),
            pltpu.VMEM((C_LEN, BPW), jnp.int32),
            pltpu.VMEM((BPW, EMBED_D), jnp.float32),
            pltpu.VMEM((BPW, EMBED_D), jnp.float32),
            pltpu.VMEM((BPW, EMBED_D), jnp.float32),
            pltpu.VMEM((BPW, EMBED_D), jnp.float32),
            pltpu.SemaphoreType.DMA((NB,)),
            pltpu.SemaphoreType.DMA((NB,)),
        ],
        compiler_params=pltpu.CompilerParams(needs_layout_passes=False),
    )
    return run(flat_ids, p_content_pad, word_embed)


def _lstm_body(emb_ref, wcat_ref, bias_ref, out_ref, h_ref, c_ref):
    t = pl.program_id(0)

    @pl.when(t == 0)
    def _init():
        h_ref[...] = jnp.zeros_like(h_ref)
        c_ref[...] = jnp.zeros_like(c_ref)
        out_ref[...] = jnp.zeros_like(out_ref)

    acc = out_ref[...]
    h = h_ref[...]
    c = c_ref[...]
    for k in range(T_BLK):
        x = emb_ref[pl.ds(k * BC, BC), :].astype(jnp.bfloat16)
        z = jnp.concatenate([x, h.astype(jnp.bfloat16)], axis=1)
        gates = (
            jnp.dot(z, wcat_ref[...], preferred_element_type=jnp.float32)
            + bias_ref[0:1, :]
        )
        th_i = jnp.tanh(gates[:, 0:EMBED_D])
        th_f = jnp.tanh(gates[:, EMBED_D:2 * EMBED_D])
        g = jnp.tanh(gates[:, 2 * EMBED_D:3 * EMBED_D])
        th_o = jnp.tanh(gates[:, 3 * EMBED_D:4 * EMBED_D])
        c = 0.5 * ((1.0 + th_f) * c + (1.0 + th_i) * g)
        h = (1.0 + th_o) * jnp.tanh(c)
        acc += h
    h_ref[...] = h
    c_ref[...] = c
    out_ref[...] = acc

    @pl.when(t == T_OUTER - 1)
    def _finish():
        out_ref[...] = acc * (1.0 / (2 * C_LEN))


def _lstm(emb_tm, wcat, bias):
    return pl.pallas_call(
        _lstm_body,
        grid=(T_OUTER,),
        in_specs=[
            pl.BlockSpec((T_BLK * BC, EMBED_D), lambda t: (t, 0)),
            pl.BlockSpec((2 * EMBED_D, 4 * EMBED_D), lambda t: (0, 0)),
            pl.BlockSpec((8, 4 * EMBED_D), lambda t: (0, 0)),
        ],
        out_specs=pl.BlockSpec((BC, EMBED_D), lambda t: (0, 0)),
        out_shape=jax.ShapeDtypeStruct((BC, EMBED_D), jnp.float32),
        scratch_shapes=[
            pltpu.VMEM((BC, EMBED_D), jnp.float32),
            pltpu.VMEM((BC, EMBED_D), jnp.float32),
        ],
        compiler_params=pltpu.CompilerParams(
            vmem_limit_bytes=100 * 1024 * 1024),
    )(emb_tm, wcat, bias)


def kernel(triple_batch, triple_index, word_embed, p_content, W_ih, W_hh,
           b_ih, b_hh):
    flat_ids = jnp.transpose(triple_batch.astype(jnp.int32)).reshape(B3)
    p_content_pad = jnp.pad(
        p_content.astype(jnp.int32), ((0, 0), (0, 128 - C_LEN)))

    gate_scale = jnp.concatenate([
        jnp.full((2 * EMBED_D,), 0.5, jnp.float32),
        jnp.ones((EMBED_D,), jnp.float32),
        jnp.full((EMBED_D,), 0.5, jnp.float32),
    ])
    wih_t = jnp.transpose(W_ih) * gate_scale[None, :]
    whh_t2 = jnp.transpose(W_hh) * gate_scale[None, :] * 0.5
    wcat = jnp.concatenate([wih_t, whh_t2], axis=0).astype(jnp.bfloat16)
    bias = jnp.broadcast_to(
        ((b_ih + b_hh) * gate_scale)[None, :], (8, 4 * EMBED_D))

    embs = [
        _sc_gather(flat_ids[c * BC:(c + 1) * BC], p_content_pad, word_embed)
        for c in range(N_CHUNK)
    ]
    outs = [_lstm(emb, wcat, bias) for emb in embs]
    out = jnp.concatenate(outs, axis=0)
    return (out[0:1024], out[1024:2048], out[2048:3072])


# speedup vs baseline: 7.4634x; 1.0659x over previous
import functools

import jax
import jax.numpy as jnp
from jax import lax
from jax.experimental import pallas as pl
from jax.experimental.pallas import tpu as pltpu
from jax.experimental.pallas import tpu_sc as plsc

EMBED_D = 128
C_LEN = 100
B3 = 3072
NUM_CORES = 2
NUM_SUBCORES = 16
NW = NUM_CORES * NUM_SUBCORES
NB = 4
TG = 2
T_BLK = 20
T_OUTER = C_LEN // T_BLK
N_CHUNK = 2
BC = B3 // N_CHUNK
BPW = BC // NW


def _sc_gather_body(ids_hbm, pcontent_hbm, wembed_hbm, emb_hbm,
                    ids_v, tok_v, tok_t, buf0, buf1, buf2, buf3, sg, ss):
    bufs = (buf0, buf1, buf2, buf3)
    wid = lax.axis_index("s") * NUM_CORES + lax.axis_index("c")
    base = wid * BPW
    pltpu.sync_copy(ids_hbm.at[pl.ds(base, BPW)], ids_v)
    pltpu.async_copy(pcontent_hbm.at[ids_v], tok_v, sg.at[0]).wait()

    iota = lax.iota(jnp.int32, 16)

    def transpose_row(t, carry):
        for i0 in range(BPW // 16):
            rows = iota + (i0 * 16)
            cols = iota * 0 + t
            tok_t[pl.ds(t * BPW + i0 * 16, 16)] = plsc.load_gather(
                tok_v, [rows, cols])
        return carry

    lax.fori_loop(0, C_LEN, transpose_row, 0)

    def gather_g(g, slot):
        pltpu.async_copy(
            wembed_hbm.at[tok_t.at[pl.ds(g * TG * BPW, TG * BPW)]],
            bufs[slot], sg.at[slot])

    def wait_gather(g, slot):
        pltpu.make_async_copy(
            wembed_hbm.at[tok_t.at[pl.ds(g * TG * BPW, TG * BPW)]],
            bufs[slot], sg.at[slot]).wait()

    def fire_scatters(g, slot):
        for i in range(TG):
            pltpu.async_copy(
                bufs[slot].at[pl.ds(i * BPW, BPW)],
                emb_hbm.at[pl.ds((g * TG + i) * BC + base, BPW)],
                ss.at[slot])

    def drain_scatter(slot):
        pltpu.make_async_copy(
            bufs[slot], emb_hbm.at[pl.ds(0, TG * BPW)], ss.at[slot]).wait()

    NG = C_LEN // TG
    for j in range(2):
        gather_g(j, j)

    def round_body(r, carry):
        for j in range(NB):
            g = r * NB + j
            wait_gather(g, j)
            fire_scatters(g, j)
            f = g + 2
            fs = (j + 2) % NB

            @pl.when(f < NG)
            def _fire():
                @pl.when(g >= 2)
                def _drain():
                    drain_scatter(fs)
                gather_g(f, fs)
        return carry

    lax.fori_loop(0, NG // NB, round_body, 0)
    for j in range(NG % NB):
        g_tail = (NG // NB) * NB + j
        wait_gather(g_tail, j)
        fire_scatters(g_tail, j)
    for j in range(NB):
        drain_scatter(j)


def _sc_gather(flat_ids, p_content_pad, word_embed):
    mesh = plsc.VectorSubcoreMesh(
        core_axis_name="c", subcore_axis_name="s",
        num_cores=NUM_CORES, num_subcores=NUM_SUBCORES)
    run = pl.kernel(
        _sc_gather_body,
        out_type=jax.ShapeDtypeStruct((C_LEN * BC, EMBED_D), jnp.float32),
        mesh=mesh,
        scratch_types=[
            pltpu.VMEM((BPW,), jnp.int32),
            pltpu.VMEM((BPW, 128), jnp.int32),
            pltpu.VMEM((C_LEN * BPW,), jnp.int32),
            pltpu.VMEM((TG * BPW, EMBED_D), jnp.float32),
            pltpu.VMEM((TG * BPW, EMBED_D), jnp.float32),
            pltpu.VMEM((TG * BPW, EMBED_D), jnp.float32),
            pltpu.VMEM((TG * BPW, EMBED_D), jnp.float32),
            pltpu.SemaphoreType.DMA((NB,)),
            pltpu.SemaphoreType.DMA((NB,)),
        ],
        compiler_params=pltpu.CompilerParams(needs_layout_passes=False),
    )
    return run(flat_ids, p_content_pad, word_embed)


def _lstm_body(emb_ref, wcat_ref, bias_ref, out_ref, h_ref, c_ref):
    t = pl.program_id(0)

    @pl.when(t == 0)
    def _init():
        h_ref[...] = jnp.zeros_like(h_ref)
        c_ref[...] = jnp.zeros_like(c_ref)
        out_ref[...] = jnp.zeros_like(out_ref)

    acc = out_ref[...]
    h = h_ref[...]
    c = c_ref[...]
    for k in range(T_BLK):
        x = emb_ref[pl.ds(k * BC, BC), :].astype(jnp.bfloat16)
        z = jnp.concatenate([x, h.astype(jnp.bfloat16)], axis=1)
        gates = (
            jnp.dot(z, wcat_ref[...], preferred_element_type=jnp.float32)
            + bias_ref[0:1, :]
        )
        th_i = jnp.tanh(gates[:, 0:EMBED_D])
        th_f = jnp.tanh(gates[:, EMBED_D:2 * EMBED_D])
        g = jnp.tanh(gates[:, 2 * EMBED_D:3 * EMBED_D])
        th_o = jnp.tanh(gates[:, 3 * EMBED_D:4 * EMBED_D])
        c = 0.5 * ((1.0 + th_f) * c + (1.0 + th_i) * g)
        h = (1.0 + th_o) * jnp.tanh(c)
        acc += h
    h_ref[...] = h
    c_ref[...] = c
    out_ref[...] = acc

    @pl.when(t == T_OUTER - 1)
    def _finish():
        out_ref[...] = acc * (1.0 / (2 * C_LEN))


def _lstm(emb_tm, wcat, bias):
    return pl.pallas_call(
        _lstm_body,
        grid=(T_OUTER,),
        in_specs=[
            pl.BlockSpec((T_BLK * BC, EMBED_D), lambda t: (t, 0)),
            pl.BlockSpec((2 * EMBED_D, 4 * EMBED_D), lambda t: (0, 0)),
            pl.BlockSpec((8, 4 * EMBED_D), lambda t: (0, 0)),
        ],
        out_specs=pl.BlockSpec((BC, EMBED_D), lambda t: (0, 0)),
        out_shape=jax.ShapeDtypeStruct((BC, EMBED_D), jnp.float32),
        scratch_shapes=[
            pltpu.VMEM((BC, EMBED_D), jnp.float32),
            pltpu.VMEM((BC, EMBED_D), jnp.float32),
        ],
        compiler_params=pltpu.CompilerParams(
            vmem_limit_bytes=100 * 1024 * 1024),
    )(emb_tm, wcat, bias)


def kernel(triple_batch, triple_index, word_embed, p_content, W_ih, W_hh,
           b_ih, b_hh):
    flat_ids = jnp.transpose(triple_batch.astype(jnp.int32)).reshape(B3)
    p_content_pad = jnp.pad(
        p_content.astype(jnp.int32), ((0, 0), (0, 128 - C_LEN)))

    gate_scale = jnp.concatenate([
        jnp.full((2 * EMBED_D,), 0.5, jnp.float32),
        jnp.ones((EMBED_D,), jnp.float32),
        jnp.full((EMBED_D,), 0.5, jnp.float32),
    ])
    wih_t = jnp.transpose(W_ih) * gate_scale[None, :]
    whh_t2 = jnp.transpose(W_hh) * gate_scale[None, :] * 0.5
    wcat = jnp.concatenate([wih_t, whh_t2], axis=0).astype(jnp.bfloat16)
    bias = jnp.broadcast_to(
        ((b_ih + b_hh) * gate_scale)[None, :], (8, 4 * EMBED_D))

    embs = [
        _sc_gather(flat_ids[c * BC:(c + 1) * BC], p_content_pad, word_embed)
        for c in range(N_CHUNK)
    ]
    outs = [_lstm(emb, wcat, bias) for emb in embs]
    out = jnp.concatenate(outs, axis=0)
    return (out[0:1024], out[1024:2048], out[2048:3072])


# speedup vs baseline: 7.7567x; 1.0393x over previous
import functools

import jax
import jax.numpy as jnp
from jax import lax
from jax.experimental import pallas as pl
from jax.experimental.pallas import tpu as pltpu
from jax.experimental.pallas import tpu_sc as plsc

EMBED_D = 128
C_LEN = 100
B3 = 3072
NUM_CORES = 2
NUM_SUBCORES = 16
NW = NUM_CORES * NUM_SUBCORES
NB = 4
TG = 4
T_BLK = 20
T_OUTER = C_LEN // T_BLK
N_CHUNK = 3
BC = B3 // N_CHUNK
BPW = BC // NW


def _sc_gather_body(ids_hbm, pcontent_hbm, wembed_hbm, emb_hbm,
                    ids_v, tok_v, tok_t, buf0, buf1, buf2, buf3, sg, ss):
    bufs = (buf0, buf1, buf2, buf3)
    wid = lax.axis_index("s") * NUM_CORES + lax.axis_index("c")
    base = wid * BPW
    pltpu.sync_copy(ids_hbm.at[pl.ds(base, BPW)], ids_v)
    pltpu.async_copy(pcontent_hbm.at[ids_v], tok_v, sg.at[0]).wait()

    iota = lax.iota(jnp.int32, 16)

    def transpose_row(t, carry):
        for i0 in range(BPW // 16):
            rows = iota + (i0 * 16)
            cols = iota * 0 + t
            tok_t[pl.ds(t * BPW + i0 * 16, 16)] = plsc.load_gather(
                tok_v, [rows, cols])
        return carry

    lax.fori_loop(0, C_LEN, transpose_row, 0)

    def gather_g(g, slot):
        pltpu.async_copy(
            wembed_hbm.at[tok_t.at[pl.ds(g * TG * BPW, TG * BPW)]],
            bufs[slot], sg.at[slot])

    def wait_gather(g, slot):
        pltpu.make_async_copy(
            wembed_hbm.at[tok_t.at[pl.ds(g * TG * BPW, TG * BPW)]],
            bufs[slot], sg.at[slot]).wait()

    def fire_scatters(g, slot):
        for i in range(TG):
            pltpu.async_copy(
                bufs[slot].at[pl.ds(i * BPW, BPW)],
                emb_hbm.at[pl.ds((g * TG + i) * BC + base, BPW)],
                ss.at[slot])

    def drain_scatter(slot):
        pltpu.make_async_copy(
            bufs[slot], emb_hbm.at[pl.ds(0, TG * BPW)], ss.at[slot]).wait()

    NG = C_LEN // TG
    for j in range(2):
        gather_g(j, j)

    def round_body(r, carry):
        for j in range(NB):
            g = r * NB + j
            wait_gather(g, j)
            fire_scatters(g, j)
            f = g + 2
            fs = (j + 2) % NB

            @pl.when(f < NG)
            def _fire():
                @pl.when(g >= 2)
                def _drain():
                    drain_scatter(fs)
                gather_g(f, fs)
        return carry

    lax.fori_loop(0, NG // NB, round_body, 0)
    for j in range(NG % NB):
        g_tail = (NG // NB) * NB + j
        wait_gather(g_tail, j)
        fire_scatters(g_tail, j)
    for j in range(NB):
        drain_scatter(j)


def _sc_gather(flat_ids, p_content_pad, word_embed):
    mesh = plsc.VectorSubcoreMesh(
        core_axis_name="c", subcore_axis_name="s",
        num_cores=NUM_CORES, num_subcores=NUM_SUBCORES)
    run = pl.kernel(
        _sc_gather_body,
        out_type=jax.ShapeDtypeStruct((C_LEN * BC, EMBED_D), jnp.float32),
        mesh=mesh,
        scratch_types=[
            pltpu.VMEM((BPW,), jnp.int32),
            pltpu.VMEM((BPW, 128), jnp.int32),
            pltpu.VMEM((C_LEN * BPW,), jnp.int32),
            pltpu.VMEM((TG * BPW, EMBED_D), jnp.float32),
            pltpu.VMEM((TG * BPW, EMBED_D), jnp.float32),
            pltpu.VMEM((TG * BPW, EMBED_D), jnp.float32),
            pltpu.VMEM((TG * BPW, EMBED_D), jnp.float32),
            pltpu.SemaphoreType.DMA((NB,)),
            pltpu.SemaphoreType.DMA((NB,)),
        ],
        compiler_params=pltpu.CompilerParams(needs_layout_passes=False),
    )
    return run(flat_ids, p_content_pad, word_embed)


def _lstm_body(emb_ref, wcat_ref, bias_ref, out_ref, h_ref, c_ref):
    t = pl.program_id(0)

    @pl.when(t == 0)
    def _init():
        h_ref[...] = jnp.zeros_like(h_ref)
        c_ref[...] = jnp.zeros_like(c_ref)
        out_ref[...] = jnp.zeros_like(out_ref)

    acc = out_ref[...]
    h = h_ref[...]
    c = c_ref[...]
    for k in range(T_BLK):
        x = emb_ref[pl.ds(k * BC, BC), :].astype(jnp.bfloat16)
        z = jnp.concatenate([x, h.astype(jnp.bfloat16)], axis=1)
        gates = (
            jnp.dot(z, wcat_ref[...], preferred_element_type=jnp.float32)
            + bias_ref[0:1, :]
        )
        th_i = jnp.tanh(gates[:, 0:EMBED_D])
        th_f = jnp.tanh(gates[:, EMBED_D:2 * EMBED_D])
        g = jnp.tanh(gates[:, 2 * EMBED_D:3 * EMBED_D])
        th_o = jnp.tanh(gates[:, 3 * EMBED_D:4 * EMBED_D])
        c = 0.5 * ((1.0 + th_f) * c + (1.0 + th_i) * g)
        h = (1.0 + th_o) * jnp.tanh(c)
        acc += h
    h_ref[...] = h
    c_ref[...] = c
    out_ref[...] = acc

    @pl.when(t == T_OUTER - 1)
    def _finish():
        out_ref[...] = acc * (1.0 / (2 * C_LEN))


def _lstm(emb_tm, wcat, bias):
    return pl.pallas_call(
        _lstm_body,
        grid=(T_OUTER,),
        in_specs=[
            pl.BlockSpec((T_BLK * BC, EMBED_D), lambda t: (t, 0)),
            pl.BlockSpec((2 * EMBED_D, 4 * EMBED_D), lambda t: (0, 0)),
            pl.BlockSpec((8, 4 * EMBED_D), lambda t: (0, 0)),
        ],
        out_specs=pl.BlockSpec((BC, EMBED_D), lambda t: (0, 0)),
        out_shape=jax.ShapeDtypeStruct((BC, EMBED_D), jnp.float32),
        scratch_shapes=[
            pltpu.VMEM((BC, EMBED_D), jnp.float32),
            pltpu.VMEM((BC, EMBED_D), jnp.float32),
        ],
        compiler_params=pltpu.CompilerParams(
            vmem_limit_bytes=100 * 1024 * 1024),
    )(emb_tm, wcat, bias)


def kernel(triple_batch, triple_index, word_embed, p_content, W_ih, W_hh,
           b_ih, b_hh):
    flat_ids = jnp.transpose(triple_batch.astype(jnp.int32)).reshape(B3)
    p_content_pad = jnp.pad(
        p_content.astype(jnp.int32), ((0, 0), (0, 128 - C_LEN)))

    gate_scale = jnp.concatenate([
        jnp.full((2 * EMBED_D,), 0.5, jnp.float32),
        jnp.ones((EMBED_D,), jnp.float32),
        jnp.full((EMBED_D,), 0.5, jnp.float32),
    ])
    wih_t = jnp.transpose(W_ih) * gate_scale[None, :]
    whh_t2 = jnp.transpose(W_hh) * gate_scale[None, :] * 0.5
    wcat = jnp.concatenate([wih_t, whh_t2], axis=0).astype(jnp.bfloat16)
    bias = jnp.broadcast_to(
        ((b_ih + b_hh) * gate_scale)[None, :], (8, 4 * EMBED_D))

    embs = [
        _sc_gather(flat_ids[c * BC:(c + 1) * BC], p_content_pad, word_embed)
        for c in range(N_CHUNK)
    ]
    outs = [_lstm(emb, wcat, bias) for emb in embs]
    return (outs[0], outs[1], outs[2])


# speedup vs baseline: 8.5397x; 1.1009x over previous
import functools

import jax
import jax.numpy as jnp
from jax import lax
from jax.experimental import pallas as pl
from jax.experimental.pallas import tpu as pltpu
from jax.experimental.pallas import tpu_sc as plsc

EMBED_D = 128
C_LEN = 100
B3 = 3072
NUM_CORES = 2
NUM_SUBCORES = 16
NW = NUM_CORES * NUM_SUBCORES
NB = 4
TG = 4
T_BLK = 20
T_OUTER = C_LEN // T_BLK
N_CHUNK = 3
BC = B3 // N_CHUNK
BPW = BC // NW


def _sc_gather_body(ids_hbm, pcontent_hbm, wembed_hbm, emb_hbm,
                    ids_v, tok_v, tok_t, buf0, buf1, buf2, buf3, sg, ss):
    bufs = (buf0, buf1, buf2, buf3)
    wid = lax.axis_index("s") * NUM_CORES + lax.axis_index("c")
    base = wid * BPW
    pltpu.sync_copy(ids_hbm.at[pl.ds(base, BPW)], ids_v)
    pltpu.async_copy(pcontent_hbm.at[ids_v], tok_v, sg.at[0]).wait()

    iota = lax.iota(jnp.int32, 16)

    def transpose_row(t, carry):
        for i0 in range(BPW // 16):
            rows = iota + (i0 * 16)
            cols = iota * 0 + t
            tok_t[pl.ds(t * BPW + i0 * 16, 16)] = plsc.load_gather(
                tok_v, [rows, cols])
        return carry

    lax.fori_loop(0, C_LEN, transpose_row, 0)

    def gather_g(g, slot):
        pltpu.async_copy(
            wembed_hbm.at[tok_t.at[pl.ds(g * TG * BPW, TG * BPW)]],
            bufs[slot], sg.at[slot])

    def wait_gather(g, slot):
        pltpu.make_async_copy(
            wembed_hbm.at[tok_t.at[pl.ds(g * TG * BPW, TG * BPW)]],
            bufs[slot], sg.at[slot]).wait()

    def fire_scatters(g, slot):
        for i in range(TG):
            pltpu.async_copy(
                bufs[slot].at[pl.ds(i * BPW, BPW)],
                emb_hbm.at[pl.ds((g * TG + i) * BC + base, BPW)],
                ss.at[slot])

    def drain_scatter(slot):
        pltpu.make_async_copy(
            bufs[slot], emb_hbm.at[pl.ds(0, TG * BPW)], ss.at[slot]).wait()

    NG = C_LEN // TG
    for j in range(2):
        gather_g(j, j)

    def round_body(r, carry):
        for j in range(NB):
            g = r * NB + j
            wait_gather(g, j)
            fire_scatters(g, j)
            f = g + 2
            fs = (j + 2) % NB

            @pl.when(f < NG)
            def _fire():
                @pl.when(g >= 2)
                def _drain():
                    drain_scatter(fs)
                gather_g(f, fs)
        return carry

    lax.fori_loop(0, NG // NB, round_body, 0)
    for j in range(NG % NB):
        g_tail = (NG // NB) * NB + j
        wait_gather(g_tail, j)
        fire_scatters(g_tail, j)
    for j in range(NB):
        drain_scatter(j)


def _sc_gather(flat_ids, p_content_pad, word_embed):
    mesh = plsc.VectorSubcoreMesh(
        core_axis_name="c", subcore_axis_name="s",
        num_cores=NUM_CORES, num_subcores=NUM_SUBCORES)
    run = pl.kernel(
        _sc_gather_body,
        out_type=jax.ShapeDtypeStruct((C_LEN * BC, EMBED_D), jnp.float32),
        mesh=mesh,
        scratch_types=[
            pltpu.VMEM((BPW,), jnp.int32),
            pltpu.VMEM((BPW, 128), jnp.int32),
            pltpu.VMEM((C_LEN * BPW,), jnp.int32),
            pltpu.VMEM((TG * BPW, EMBED_D), jnp.float32),
            pltpu.VMEM((TG * BPW, EMBED_D), jnp.float32),
            pltpu.VMEM((TG * BPW, EMBED_D), jnp.float32),
            pltpu.VMEM((TG * BPW, EMBED_D), jnp.float32),
            pltpu.SemaphoreType.DMA((NB,)),
            pltpu.SemaphoreType.DMA((NB,)),
        ],
        compiler_params=pltpu.CompilerParams(needs_layout_passes=False),
    )
    return run(flat_ids, p_content_pad, word_embed)


def _pad_body(src_ref, dst_ref):
    dst_ref[:, 0:C_LEN] = src_ref[...]


def _pad_tokens(p_content_i32):
    return pl.pallas_call(
        _pad_body,
        grid=(10,),
        in_specs=[pl.BlockSpec((2000, C_LEN), lambda i: (i, 0))],
        out_specs=pl.BlockSpec((2000, 128), lambda i: (i, 0)),
        out_shape=jax.ShapeDtypeStruct((20000, 128), jnp.int32),
    )(p_content_i32)


def _lstm_body(emb_ref, wcat_ref, bias_ref, out_ref, h_ref, c_ref):
    t = pl.program_id(0)

    @pl.when(t == 0)
    def _init():
        h_ref[...] = jnp.zeros_like(h_ref)
        c_ref[...] = jnp.zeros_like(c_ref)
        out_ref[...] = jnp.zeros_like(out_ref)

    acc = out_ref[...]
    h = h_ref[...]
    c = c_ref[...]
    for k in range(T_BLK):
        x = emb_ref[pl.ds(k * BC, BC), :].astype(jnp.bfloat16)
        z = jnp.concatenate([x, h.astype(jnp.bfloat16)], axis=1)
        gates = (
            jnp.dot(z, wcat_ref[...], preferred_element_type=jnp.float32)
            + bias_ref[0:1, :]
        )
        th_i = jnp.tanh(gates[:, 0:EMBED_D])
        th_f = jnp.tanh(gates[:, EMBED_D:2 * EMBED_D])
        g = jnp.tanh(gates[:, 2 * EMBED_D:3 * EMBED_D])
        th_o = jnp.tanh(gates[:, 3 * EMBED_D:4 * EMBED_D])
        c = 0.5 * ((1.0 + th_f) * c + (1.0 + th_i) * g)
        h = (1.0 + th_o) * jnp.tanh(c)
        acc += h
    h_ref[...] = h
    c_ref[...] = c
    out_ref[...] = acc

    @pl.when(t == T_OUTER - 1)
    def _finish():
        out_ref[...] = acc * (1.0 / (2 * C_LEN))


def _lstm(emb_tm, wcat, bias):
    return pl.pallas_call(
        _lstm_body,
        grid=(T_OUTER,),
        in_specs=[
            pl.BlockSpec((T_BLK * BC, EMBED_D), lambda t: (t, 0)),
            pl.BlockSpec((2 * EMBED_D, 4 * EMBED_D), lambda t: (0, 0)),
            pl.BlockSpec((8, 4 * EMBED_D), lambda t: (0, 0)),
        ],
        out_specs=pl.BlockSpec((BC, EMBED_D), lambda t: (0, 0)),
        out_shape=jax.ShapeDtypeStruct((BC, EMBED_D), jnp.float32),
        scratch_shapes=[
            pltpu.VMEM((BC, EMBED_D), jnp.float32),
            pltpu.VMEM((BC, EMBED_D), jnp.float32),
        ],
        compiler_params=pltpu.CompilerParams(
            vmem_limit_bytes=100 * 1024 * 1024),
    )(emb_tm, wcat, bias)


def kernel(triple_batch, triple_index, word_embed, p_content, W_ih, W_hh,
           b_ih, b_hh):
    flat_ids = jnp.transpose(triple_batch.astype(jnp.int32)).reshape(B3)
    p_content_pad = _pad_tokens(p_content.astype(jnp.int32))

    gate_scale = jnp.concatenate([
        jnp.full((2 * EMBED_D,), 0.5, jnp.float32),
        jnp.ones((EMBED_D,), jnp.float32),
        jnp.full((EMBED_D,), 0.5, jnp.float32),
    ])
    wih_t = jnp.transpose(W_ih) * gate_scale[None, :]
    whh_t2 = jnp.transpose(W_hh) * gate_scale[None, :] * 0.5
    wcat = jnp.concatenate([wih_t, whh_t2], axis=0).astype(jnp.bfloat16)
    bias = jnp.broadcast_to(
        ((b_ih + b_hh) * gate_scale)[None, :], (8, 4 * EMBED_D))

    embs = [
        _sc_gather(flat_ids[c * BC:(c + 1) * BC], p_content_pad, word_embed)
        for c in range(N_CHUNK)
    ]
    outs = [_lstm(emb, wcat, bias) for emb in embs]
    return (outs[0], outs[1], outs[2])
